# Initial kernel scaffold; baseline (speedup 1.0000x reference)
import jax
import jax.numpy as jnp
from jax.experimental import pallas as pl


def kernel(x, onset_table, medial_table, nucleus_table, coda_table):
    raise NotImplementedError("write your pallas kernel here")


# speedup vs baseline: 4.7141x; 4.7141x over previous
import functools

import jax
import jax.numpy as jnp
from jax import lax
from jax.experimental import pallas as pl
from jax.experimental.pallas import tpu as pltpu
from jax.experimental.pallas import tpu_sc as plsc

B, S, D = 4096, 200, 128
BS = B * S
NC, NS, L = 2, 16, 16
NW = NC * NS
PW = BS // NW
CH = 128
NIT = PW // CH


def _body(xf, t0, t1, t2, t3, out, xbuf, i0, i1, i2, i3, r0, r1, r2, r3, sem):
    wid = lax.axis_index("s") * NC + lax.axis_index("c")
    base = wid * PW
    ibufs = (i0, i1, i2, i3)
    rbufs = (r0, r1, r2, r3)
    tabs = (t0, t1, t2, t3)

    def chunk(i, _):
        cbase = base + i * CH
        pltpu.sync_copy(xf.at[pl.ds(cbase * 4, CH * 4)], xbuf)

        def deint(k, _):
            lanes = lax.iota(jnp.int32, L) * 4 + k * (4 * L)
            for t in range(4):
                v = plsc.load_gather(xbuf, [lanes + t])
                ibufs[t][pl.ds(k * L, L)] = v
            return 0

        lax.fori_loop(0, CH // L, deint, 0)

        cps = [
            pltpu.async_copy(tabs[t].at[ibufs[t]], rbufs[t], sem)
            for t in range(4)
        ]
        for c in cps:
            c.wait()

        def accum(j, _):
            for l in range(D // L):
                s = pl.ds(l * L, L)
                v = r0[j, s] + r1[j, s] + r2[j, s] + r3[j, s]
                r0[j, s] = v
            return 0

        lax.fori_loop(0, CH, accum, 0)
        pltpu.sync_copy(r0, out.at[pl.ds(cbase, CH)])
        return 0

    lax.fori_loop(0, NIT, chunk, 0)


@jax.jit
def kernel(x, onset_table, medial_table, nucleus_table, coda_table):
    xf = x.reshape(-1)
    mesh = plsc.VectorSubcoreMesh(core_axis_name="c", subcore_axis_name="s")
    kfn = pl.kernel(
        _body,
        out_type=jax.ShapeDtypeStruct((BS, D), jnp.float32),
        mesh=mesh,
        compiler_params=pltpu.CompilerParams(needs_layout_passes=False),
        scratch_types=[
            pltpu.VMEM((CH * 4,), jnp.int32),
            pltpu.VMEM((CH,), jnp.int32),
            pltpu.VMEM((CH,), jnp.int32),
            pltpu.VMEM((CH,), jnp.int32),
            pltpu.VMEM((CH,), jnp.int32),
            pltpu.VMEM((CH, D), jnp.float32),
            pltpu.VMEM((CH, D), jnp.float32),
            pltpu.VMEM((CH, D), jnp.float32),
            pltpu.VMEM((CH, D), jnp.float32),
            pltpu.SemaphoreType.DMA,
        ],
    )
    out = kfn(xf, onset_table, medial_table, nucleus_table, coda_table)
    return out.reshape(B, S, D)


# speedup vs baseline: 5.0860x; 1.0789x over previous
import functools

import jax
import jax.numpy as jnp
from jax import lax
from jax.experimental import pallas as pl
from jax.experimental.pallas import tpu as pltpu
from jax.experimental.pallas import tpu_sc as plsc

B, S, D = 4096, 200, 128
BS = B * S
NC, NS, L = 2, 16, 16
NW = NC * NS
PW = BS // NW
CH = 128
NIT = PW // CH


def _body(xf, t0, t1, t2, t3, out, xbuf, i0, i1, i2, i3, r0, r1, r2, r3,
          s0, s1, s2, s3, sem):
    sid = lax.axis_index("s")
    wid = sid * NC + lax.axis_index("c")
    base = wid * PW
    ibufs = (i0, i1, i2, i3)
    rbufs = (r0, r1, r2, r3)
    tabs = (t0, t1, t2, t3)
    shtabs = (s0, s1, s2, s3)

    bounds = (0, 256, 512, 768, 1000)
    for t in range(4):
        for q in range(4):
            @pl.when(sid == q * 4 + t)
            def _(t=t, q=q):
                lo, hi = bounds[q], bounds[q + 1]
                pltpu.sync_copy(
                    tabs[t].at[pl.ds(lo, hi - lo)],
                    shtabs[t].at[pl.ds(lo, hi - lo)],
                )
    plsc.subcore_barrier()

    def chunk(i, _):
        cbase = base + i * CH
        pltpu.sync_copy(xf.at[pl.ds(cbase * 4, CH * 4)], xbuf)

        def deint(k, _):
            lanes = lax.iota(jnp.int32, L) * 4 + k * (4 * L)
            for t in range(4):
                v = plsc.load_gather(xbuf, [lanes + t])
                ibufs[t][pl.ds(k * L, L)] = v
            return 0

        lax.fori_loop(0, CH // L, deint, 0)

        cps = [
            pltpu.async_copy(shtabs[t].at[ibufs[t]], rbufs[t], sem)
            for t in range(4)
        ]
        for c in cps:
            c.wait()

        def accum(j, _):
            for l in range(D // L):
                s = pl.ds(l * L, L)
                v = r0[j, s] + r1[j, s] + r2[j, s] + r3[j, s]
                r0[j, s] = v
            return 0

        lax.fori_loop(0, CH, accum, 0)
        pltpu.sync_copy(r0, out.at[pl.ds(cbase, CH)])
        return 0

    lax.fori_loop(0, NIT, chunk, 0)


@jax.jit
def kernel(x, onset_table, medial_table, nucleus_table, coda_table):
    xf = x.reshape(-1)
    mesh = plsc.VectorSubcoreMesh(core_axis_name="c", subcore_axis_name="s")
    kfn = pl.kernel(
        _body,
        out_type=jax.ShapeDtypeStruct((BS, D), jnp.float32),
        mesh=mesh,
        compiler_params=pltpu.CompilerParams(needs_layout_passes=False),
        scratch_types=[
            pltpu.VMEM((CH * 4,), jnp.int32),
            pltpu.VMEM((CH,), jnp.int32),
            pltpu.VMEM((CH,), jnp.int32),
            pltpu.VMEM((CH,), jnp.int32),
            pltpu.VMEM((CH,), jnp.int32),
            pltpu.VMEM((CH, D), jnp.float32),
            pltpu.VMEM((CH, D), jnp.float32),
            pltpu.VMEM((CH, D), jnp.float32),
            pltpu.VMEM((CH, D), jnp.float32),
            pltpu.VMEM_SHARED((1000, D), jnp.float32),
            pltpu.VMEM_SHARED((1000, D), jnp.float32),
            pltpu.VMEM_SHARED((1000, D), jnp.float32),
            pltpu.VMEM_SHARED((1000, D), jnp.float32),
            pltpu.SemaphoreType.DMA,
        ],
    )
    out = kfn(xf, onset_table, medial_table, nucleus_table, coda_table)
    return out.reshape(B, S, D)
